# baseline (speedup 1.0000x reference)
import functools
import jax
import jax.numpy as jnp
from jax import lax
from jax.experimental import pallas as pl
from jax.experimental.pallas import tpu as pltpu
from jax.experimental.pallas import tpu_sc as plsc


ROWS = 4096
COLS = 20
VOCAB = 1000
CV = COLS * VOCAB
NW = 32
VB = 8
NCHUNKS = CV // VB
PER_W = -(-NCHUNKS // NW)
LANES = 16


def _sc_kernel_body(xt_hbm, out_hbm, buf, xcol):
    cid = lax.axis_index("c")
    sid = lax.axis_index("s")
    wid = sid * 2 + cid

    ones = jnp.full((LANES,), 1, jnp.int32)
    zeros = jnp.zeros((LANES,), jnp.int32)

    def _turn(k, _):
        t = wid + k * NW

        @pl.when(t < NCHUNKS)
        def _do():
            g0 = t * VB
            c = g0 // VOCAB
            v0 = g0 % VOCAB
            pltpu.sync_copy(xt_hbm.at[c], xcol)
            vvs = [jnp.full((LANES,), v0 + row, jnp.int32)
                   for row in range(VB)]

            def _grp(j, _):
                xi = xcol[pl.ds(j * LANES, LANES)]
                for row in range(VB):
                    buf[row, pl.ds(j * LANES, LANES)] = jnp.where(
                        xi == vvs[row], ones, zeros)
                return 0

            lax.fori_loop(0, ROWS // LANES, _grp, 0)
            pltpu.sync_copy(buf, out_hbm.at[pl.ds(g0, VB)])

        return 0

    lax.fori_loop(0, PER_W, _turn, 0)


def _make_sc_call():
    mesh = plsc.VectorSubcoreMesh(core_axis_name="c", subcore_axis_name="s")
    return functools.partial(
        pl.kernel,
        mesh=mesh,
        out_type=jax.ShapeDtypeStruct((CV, ROWS), jnp.int32),
        scratch_types=[
            pltpu.VMEM((VB, ROWS), jnp.int32),
            pltpu.VMEM((ROWS,), jnp.int32),
        ],
    )(_sc_kernel_body)


_sc_call = _make_sc_call()


def kernel(x):
    xt = x.astype(jnp.int32).T
    out2d = _sc_call(xt)
    return jnp.transpose(out2d.reshape(COLS, VOCAB, ROWS), (2, 0, 1))


# speedup vs baseline: 3.5894x; 3.5894x over previous
import jax
import jax.numpy as jnp
from jax import lax
from jax.experimental import pallas as pl


ROWS = 4096
COLS = 20
VOCAB = 1000
VB = 128


def _onehot_block(x_ref, out_ref):
    c = pl.program_id(0)
    v0 = pl.program_id(1) * VB
    xv = x_ref[pl.ds(c, 1), :][:, None, :]
    iota = v0 + lax.broadcasted_iota(jnp.int32, (1, VB, ROWS), 1)
    out_ref[...] = (xv == iota).astype(jnp.int32)


def kernel(x):
    xt = x.astype(jnp.int32).T
    out_t = pl.pallas_call(
        _onehot_block,
        grid=(COLS, pl.cdiv(VOCAB, VB)),
        in_specs=[pl.BlockSpec((COLS, ROWS), lambda c, v: (0, 0))],
        out_specs=pl.BlockSpec((1, VB, ROWS), lambda c, v: (c, v, 0)),
        out_shape=jax.ShapeDtypeStruct((COLS, VOCAB, ROWS), jnp.int32),
    )(xt)
    return jnp.transpose(out_t, (2, 0, 1))


# speedup vs baseline: 4.4562x; 1.2415x over previous
import jax
import jax.numpy as jnp
from jax import lax
from jax.experimental import pallas as pl


ROWS = 4096
COLS = 20
VOCAB = 1000
VB = 200


def _onehot_block(x_ref, out_ref):
    c = pl.program_id(0)
    v0 = pl.program_id(1) * VB
    xv = x_ref[pl.ds(c, 1), :][:, None, :]
    iota = v0 + lax.broadcasted_iota(jnp.int32, (1, VB, ROWS), 1)
    out_ref[...] = (xv == iota).astype(jnp.int32)


def kernel(x):
    xt = x.astype(jnp.int32).T
    out_t = pl.pallas_call(
        _onehot_block,
        grid=(COLS, VOCAB // VB),
        in_specs=[pl.BlockSpec((COLS, ROWS), lambda c, v: (0, 0))],
        out_specs=pl.BlockSpec((1, VB, ROWS), lambda c, v: (c, v, 0)),
        out_shape=jax.ShapeDtypeStruct((COLS, VOCAB, ROWS), jnp.int32),
    )(xt)
    return jnp.transpose(out_t, (2, 0, 1))
